# baseline (device time: 31182 ns/iter reference)
import jax
import jax.numpy as jnp
from jax import lax
from jax.experimental import pallas as pl
from jax.experimental.pallas import tpu as pltpu

N_DEV = 8
N_CHUNK = 4


def kernel(x, w_mat):
    m_per, k = x.shape
    _, n = w_mat.shape
    n_per = n // N_DEV
    nc = n // N_CHUNK

    def body(x_hbm, w_hbm, out_ref, xf_ref, wf_ref, y_ref,
             xload_sem, load_sems, send_sems, recv_sems):
        my = lax.axis_index("i")
        q = lax.div(my, 2)

        barrier_sem = pltpu.get_barrier_semaphore()
        for d in range(1, N_DEV):
            pl.semaphore_signal(
                barrier_sem, inc=1,
                device_id=(lax.rem(my + d, N_DEV),),
                device_id_type=pl.DeviceIdType.MESH)

        x_cp = pltpu.make_async_copy(x_hbm, xf_ref, xload_sem)
        x_cp.start()

        def w_load(j, slot):
            return pltpu.make_async_copy(
                w_hbm.at[:, pl.ds(j * nc, nc)],
                wf_ref.at[slot],
                load_sems.at[slot],
            )

        def chunk_j(c):
            return lax.rem(q + 1 + c, N_CHUNK)

        w_load(chunk_j(0), 0).start()
        x_cp.wait()
        x_val = xf_ref[:, :].astype(jnp.bfloat16)
        for c in range(N_CHUNK):
            slot = c % 2
            w_load(chunk_j(c), slot).wait()
            if c + 1 < N_CHUNK:
                w_load(chunk_j(c + 1), (c + 1) % 2).start()
            yc = jnp.dot(x_val, wf_ref[slot].astype(jnp.bfloat16),
                         preferred_element_type=jnp.float32)
            yc = (yc * jax.nn.sigmoid(yc)).astype(jnp.bfloat16)
            if c == 0:
                pl.semaphore_wait(barrier_sem, N_DEV - 1)
            for h in range(2):
                t = 2 * chunk_j(c) + h
                y_ref[t] = yc[:, h * n_per:(h + 1) * n_per]

                @pl.when(t != my)
                def _():
                    rdma = pltpu.make_async_remote_copy(
                        src_ref=y_ref.at[t],
                        dst_ref=out_ref.at[pl.ds(my * m_per, m_per), :],
                        send_sem=send_sems.at[t],
                        recv_sem=recv_sems.at[my],
                        device_id=(t,),
                        device_id_type=pl.DeviceIdType.MESH,
                    )
                    rdma.start()

        out_ref[pl.ds(my * m_per, m_per), :] = y_ref[my]

        for s in range(N_DEV):
            @pl.when(s != my)
            def _():
                recv = pltpu.make_async_remote_copy(
                    src_ref=y_ref.at[s],
                    dst_ref=out_ref.at[pl.ds(s * m_per, m_per), :],
                    send_sem=send_sems.at[s],
                    recv_sem=recv_sems.at[s],
                    device_id=(my,),
                    device_id_type=pl.DeviceIdType.MESH,
                )
                recv.wait_recv()

        for u in range(N_DEV):
            @pl.when(u != my)
            def _():
                snd = pltpu.make_async_remote_copy(
                    src_ref=y_ref.at[u],
                    dst_ref=out_ref.at[pl.ds(u * m_per, m_per), :],
                    send_sem=send_sems.at[u],
                    recv_sem=recv_sems.at[u],
                    device_id=(my,),
                    device_id_type=pl.DeviceIdType.MESH,
                )
                snd.wait_send()

    out_shape = jax.ShapeDtypeStruct((N_DEV * m_per, n_per), jnp.bfloat16)
    return pl.pallas_call(
        body,
        out_shape=out_shape,
        in_specs=[
            pl.BlockSpec(memory_space=pl.ANY),
            pl.BlockSpec(memory_space=pl.ANY),
        ],
        out_specs=pl.BlockSpec(memory_space=pltpu.VMEM),
        scratch_shapes=[
            pltpu.VMEM((m_per, k), jnp.float32),
            pltpu.VMEM((2, k, nc), jnp.float32),
            pltpu.VMEM((N_DEV, m_per, n_per), jnp.bfloat16),
            pltpu.SemaphoreType.DMA,
            pltpu.SemaphoreType.DMA((2,)),
            pltpu.SemaphoreType.DMA((N_DEV,)),
            pltpu.SemaphoreType.DMA((N_DEV,)),
        ],
        compiler_params=pltpu.CompilerParams(
            vmem_limit_bytes=100 * 1024 * 1024,
            collective_id=0,
        ),
    )(x, w_mat)
